# baseline (speedup 1.0000x reference)
import functools

import jax
import jax.numpy as jnp
from jax import lax
from jax.experimental import pallas as pl
from jax.experimental.pallas import tpu as pltpu
from jax.experimental.pallas import tpu_sc as plsc

_VOCAB = 1001
_OOV = 100
_TBL = _VOCAB + _OOV
_B, _W = 4096, 50
_NC = 2
_NS = 16
_NW = _NC * _NS
_ROWS = _B // _NW
_CHUNK = _ROWS * _W
_L = 16


def _sc_body(tok_hbm, tbl_hbm, out_hbm, tok_v, out_v, tbl_v):
    wid = lax.axis_index("s") * _NC + lax.axis_index("c")
    pltpu.sync_copy(tbl_hbm, tbl_v)
    pltpu.sync_copy(tok_hbm.reshape(_NW, _ROWS, _W).at[wid], tok_v)

    lane = lax.iota(jnp.int32, _L)

    @plsc.parallel_loop(0, _CHUNK // _L, unroll=2)
    def _loop(i):
        flat = i * _L
        r0 = flat // _W
        c0 = flat - r0 * _W
        col = c0 + lane
        wrap = col >= _W
        row = jnp.where(wrap, r0 + 1, r0)
        col = jnp.where(wrap, col - _W, col)
        tok = plsc.load_gather(tok_v, [row, col])
        q = (tok.astype(jnp.float32) * 0.01).astype(jnp.int32)
        idx = jnp.where(tok < _VOCAB, tok, _VOCAB + tok - q * _OOV)
        plsc.store_scatter(out_v, [row, col], plsc.load_gather(tbl_v, [idx]))
    pltpu.sync_copy(out_v, out_hbm.reshape(_NW, _ROWS, _W).at[wid])


@jax.jit
def kernel(tokens, table):
    mesh = plsc.VectorSubcoreMesh(core_axis_name="c", subcore_axis_name="s")
    out = pl.kernel(
        _sc_body,
        out_type=jax.ShapeDtypeStruct((_B, _W), jnp.float32),
        mesh=mesh,
        compiler_params=pltpu.CompilerParams(needs_layout_passes=False),
        scratch_types=[
            pltpu.VMEM((_ROWS, _W), jnp.int32),
            pltpu.VMEM((_ROWS, _W), jnp.float32),
            pltpu.VMEM((_TBL,), jnp.float32),
        ],
    )(tokens, table)
    return out


# speedup vs baseline: 1.0428x; 1.0428x over previous
import functools

import jax
import jax.numpy as jnp
from jax import lax
from jax.experimental import pallas as pl
from jax.experimental.pallas import tpu as pltpu
from jax.experimental.pallas import tpu_sc as plsc

_VOCAB = 1001
_OOV = 100
_TBL = _VOCAB + _OOV
_B, _W = 4096, 50
_NC = 2
_NS = 16
_NW = _NC * _NS
_ROWS = _B // _NW
_CHUNK = _ROWS * _W
_L = 16


_HALF = _ROWS // 2


def _sc_body(tok_hbm, tbl_hbm, out_hbm, tok_v, out_v, tbl_v, sem_t, sem_k, sem_o):
    wid = lax.axis_index("s") * _NC + lax.axis_index("c")
    ct = pltpu.async_copy(tbl_hbm, tbl_v, sem_t)
    ck = pltpu.async_copy(tok_hbm.reshape(_NW, _ROWS, _W).at[wid], tok_v, sem_k)
    ck.wait()
    ct.wait()

    lane = lax.iota(jnp.int32, _L)

    def make_loop(lo, hi):
        @plsc.parallel_loop(lo, hi, unroll=4)
        def _loop(i):
            flat = i * _L
            r0 = flat // _W
            c0 = flat - r0 * _W
            col = c0 + lane
            wrap = col >= _W
            row = jnp.where(wrap, r0 + 1, r0)
            col = jnp.where(wrap, col - _W, col)
            tok = plsc.load_gather(tok_v, [row, col])
            q = (tok.astype(jnp.float32) * 0.01).astype(jnp.int32)
            idx = jnp.where(tok < _VOCAB, tok, _VOCAB + tok - q * _OOV)
            plsc.store_scatter(out_v, [row, col], plsc.load_gather(tbl_v, [idx]))

    out_slab = out_hbm.reshape(_NW, _ROWS, _W).at[wid]
    make_loop(0, _HALF * _W // _L)
    co = pltpu.async_copy(out_v.at[pl.ds(0, _HALF)], out_slab.at[pl.ds(0, _HALF)],
                          sem_o)
    make_loop(_HALF * _W // _L, _CHUNK // _L)
    co.wait()
    pltpu.sync_copy(out_v.at[pl.ds(_HALF, _ROWS - _HALF)],
                    out_slab.at[pl.ds(_HALF, _ROWS - _HALF)])


@jax.jit
def kernel(tokens, table):
    mesh = plsc.VectorSubcoreMesh(core_axis_name="c", subcore_axis_name="s")
    out = pl.kernel(
        _sc_body,
        out_type=jax.ShapeDtypeStruct((_B, _W), jnp.float32),
        mesh=mesh,
        compiler_params=pltpu.CompilerParams(needs_layout_passes=False),
        scratch_types=[
            pltpu.VMEM((_ROWS, _W), jnp.int32),
            pltpu.VMEM((_ROWS, _W), jnp.float32),
            pltpu.VMEM((_TBL,), jnp.float32),
            pltpu.SemaphoreType.DMA,
            pltpu.SemaphoreType.DMA,
            pltpu.SemaphoreType.DMA,
        ],
    )(tokens, table)
    return out
